# baseline (speedup 1.0000x reference)
import functools
import jax
import jax.numpy as jnp
from jax import lax
from jax.experimental import pallas as pl
from jax.experimental.pallas import tpu as pltpu
from jax.experimental.pallas import tpu_sc as plsc

B = 8192
D_IN = 768
D_LATENT = 16384
K = 64

NW = 32
NC = 2
NCHUNK = 2
BC = B // NCHUNK
RPW = BC // NW
NV = D_LATENT // 16
NBIN = 64


def _enc_body(x_ref, bpre_ref, encW_ref, encb_ref, a_ref):
    x0 = x_ref[...] - bpre_ref[...][None, :]
    acc = jax.lax.dot_general(
        x0, encW_ref[...], (((1,), (1,)), ((), ())),
        preferred_element_type=jnp.float32,
    )
    a_ref[...] = acc + encb_ref[...][None, :]


def _encode(x, b_pre, enc_W, enc_b):
    BB, LB = 1024, 2048
    grid = (D_LATENT // LB, BC // BB)
    return pl.pallas_call(
        _enc_body,
        grid=grid,
        in_specs=[
            pl.BlockSpec((BB, D_IN), lambda j, i: (i, 0)),
            pl.BlockSpec((D_IN,), lambda j, i: (0,)),
            pl.BlockSpec((LB, D_IN), lambda j, i: (j, 0)),
            pl.BlockSpec((LB,), lambda j, i: (j,)),
        ],
        out_specs=pl.BlockSpec((BB, LB), lambda j, i: (i, j)),
        out_shape=jax.ShapeDtypeStruct((BC, D_LATENT), jnp.float32),
    )(x, b_pre, enc_W, enc_b)


def _sc_body(a_hbm, thr_hbm, rowbuf, hist, cand, thrbuf, sem0, sem1):
    wid = lax.axis_index("s") * NC + lax.axis_index("c")
    base = wid * RPW
    lane = lax.iota(jnp.int32, 16)
    ones16 = jnp.ones((16,), jnp.int32)
    zeros16 = jnp.zeros((16,), jnp.int32)
    mask_all = lane < 16
    lane_off = lane * 129

    @plsc.parallel_loop(0, 130, unroll=4)
    def _(k):
        hist[pl.ds(k * 16, 16)] = zeros16

    def process_row(buf_slot, row_idx):
        @plsc.parallel_loop(0, NV, unroll=16)
        def _(v):
            x = rowbuf[buf_slot, pl.ds(v * 16, 16)]
            bits = lax.bitcast_convert_type(x, jnp.int32)
            idx = lax.shift_right_logical(bits, 25) + lane_off
            plsc.addupdate_scatter(hist, [idx], ones16, mask=mask_all)

        run = jnp.int32(0)
        bstar = jnp.int32(0)
        cnt_above = jnp.int32(0)
        found = jnp.bool_(False)
        for c in range(NBIN // 16 - 1, -1, -1):
            acc = zeros16
            for l in range(16):
                acc = acc + hist[pl.ds(l * 129 + c * 16, 16)]
                hist[pl.ds(l * 129 + c * 16, 16)] = zeros16
            s = jnp.sum(acc)
            rc = plsc.cumsum(lax.rev(acc, (0,)))
            tot = rc + run
            below = tot < K
            p_cnt = jnp.sum(below.astype(jnp.int32))
            hit = jnp.logical_and(jnp.logical_not(found),
                                  jnp.logical_and(run + s >= K, p_cnt < 16))
            d_here = c * 16 + 15 - p_cnt
            ca_here = jnp.maximum(run, jnp.max(jnp.where(below, tot, 0)))
            bstar = jnp.where(hit, d_here, bstar)
            cnt_above = jnp.where(hit, ca_here, cnt_above)
            found = jnp.logical_or(found, hit)
            run = run + s

        lo = jnp.maximum(bstar << 25, 1)
        hi = jnp.where(bstar < (NBIN - 1), (bstar + 1) << 25,
                       jnp.int32(0x7FFFFFFF))
        cnt_above = jnp.where(found, cnt_above, run)
        k_rem = K - cnt_above

        def compact(v, off16):
            x = rowbuf[buf_slot, pl.ds(v * 16, 16)]
            bits = lax.bitcast_convert_type(x, jnp.int32)
            m = jnp.logical_and(bits >= lo, bits < hi)
            plsc.store_scatter(cand, [off16 + lane], bits, mask=m)
            return off16 + jnp.where(m, 16, 0)

        off16 = plsc.parallel_loop(0, NV, unroll=16, carry=zeros16)(compact)
        valid_n = lax.shift_right_logical(off16, 4)
        nmax = jnp.max(valid_n)
        k_rem_s = zeros16 + k_rem

        t = zeros16 + (bstar << 25)
        for b in range(24, -1, -1):
            candt = t | (1 << b)

            def count_vreg(i, acc):
                cvals = cand[pl.ds(i * 16, 16)]
                ok = jnp.logical_and(cvals >= candt, valid_n > i)
                return acc + plsc.all_reduce_population_count(ok)

            cnt = plsc.parallel_loop(0, nmax, unroll=2, carry=zeros16)(count_vreg)
            t = jnp.where(cnt >= k_rem_s, candt, t)

        tval = lax.bitcast_convert_type(jnp.maximum(t, 1), jnp.float32)
        plsc.store_scatter(thrbuf, [zeros16 + row_idx], tval, mask=lane == 0)

    pltpu.async_copy(a_hbm.at[base], rowbuf.at[0], sem0)
    pltpu.async_copy(a_hbm.at[base + 1], rowbuf.at[1], sem1)

    @pl.loop(0, RPW // 2)
    def _(g):
        r0 = base + 2 * g
        pltpu.make_async_copy(a_hbm.at[r0], rowbuf.at[0], sem0).wait()
        process_row(0, 2 * g)

        @pl.when(g < RPW // 2 - 1)
        def _():
            pltpu.async_copy(a_hbm.at[r0 + 2], rowbuf.at[0], sem0)

        pltpu.make_async_copy(a_hbm.at[r0 + 1], rowbuf.at[1], sem1).wait()
        process_row(1, 2 * g + 1)

        @pl.when(g < RPW // 2 - 1)
        def _():
            pltpu.async_copy(a_hbm.at[r0 + 3], rowbuf.at[1], sem1)

    pltpu.sync_copy(thrbuf, thr_hbm.at[pl.ds(base, RPW)])


def _sc_select(a):
    mesh = plsc.VectorSubcoreMesh(
        core_axis_name="c", subcore_axis_name="s",
        num_cores=NC, num_subcores=NW // NC,
    )
    kern = pl.kernel(
        _sc_body,
        out_type=jax.ShapeDtypeStruct((BC,), jnp.float32),
        mesh=mesh,
        compiler_params=pltpu.CompilerParams(needs_layout_passes=False),
        scratch_types=[
            pltpu.VMEM((2, D_LATENT), jnp.float32),
            pltpu.VMEM((2080,), jnp.int32),
            pltpu.VMEM((D_LATENT + 16,), jnp.int32),
            pltpu.VMEM((RPW,), jnp.float32),
            pltpu.SemaphoreType.DMA,
            pltpu.SemaphoreType.DMA,
        ],
    )
    return kern(a)


def _dec_body(a_ref, thr_ref, decW_ref, decb_ref, o_ref, z_ref):
    j = pl.program_id(1)
    z = jnp.where(a_ref[...] >= thr_ref[...], a_ref[...], 0.0)
    z_ref[...] = z
    acc = jax.lax.dot_general(
        z, decW_ref[...], (((1,), (1,)), ((), ())),
        preferred_element_type=jnp.float32,
    )

    @pl.when(j == 0)
    def _():
        o_ref[...] = acc + decb_ref[...][None, :]

    @pl.when(j > 0)
    def _():
        o_ref[...] = o_ref[...] + acc


def _decode(a, thr, dec_W, dec_b):
    BB, LB = 1024, 1024
    grid = (BC // BB, D_LATENT // LB)
    return pl.pallas_call(
        _dec_body,
        grid=grid,
        in_specs=[
            pl.BlockSpec((BB, LB), lambda i, j: (i, j)),
            pl.BlockSpec((BB, 1), lambda i, j: (i, 0)),
            pl.BlockSpec((D_IN, LB), lambda i, j: (0, j)),
            pl.BlockSpec((D_IN,), lambda i, j: (0,)),
        ],
        out_specs=[
            pl.BlockSpec((BB, D_IN), lambda i, j: (i, 0)),
            pl.BlockSpec((BB, LB), lambda i, j: (i, j)),
        ],
        out_shape=[
            jax.ShapeDtypeStruct((BC, D_IN), jnp.float32),
            jax.ShapeDtypeStruct((BC, D_LATENT), jnp.float32),
        ],
    )(a, thr, dec_W, dec_b)


def kernel(x, b_pre, enc_W, enc_b, dec_W, dec_b):
    a_c = []
    thr_c = []
    for i in range(NCHUNK):
        a_c.append(_encode(x[i * BC:(i + 1) * BC], b_pre, enc_W, enc_b))
        thr_c.append(_sc_select(a_c[i]))
    outs = [_decode(a_c[i], thr_c[i].reshape(BC, 1), dec_W, dec_b)
            for i in range(NCHUNK)]
    x_hat = jnp.concatenate([o[0] for o in outs], axis=0)
    z = jnp.concatenate([o[1] for o in outs], axis=0)
    a = jnp.concatenate(a_c, axis=0)
    return (x_hat, z, a)


# speedup vs baseline: 1.2208x; 1.2208x over previous
import functools
import jax
import jax.numpy as jnp
from jax import lax
from jax.experimental import pallas as pl
from jax.experimental.pallas import tpu as pltpu
from jax.experimental.pallas import tpu_sc as plsc

B = 8192
D_IN = 768
D_LATENT = 16384
K = 64

NW = 32
NC = 2
NCHUNK = 1
BC = B // NCHUNK
RPW = BC // NW
NV = D_LATENT // 16
NBIN = 64


def _enc_body(x_ref, bpre_ref, encW_ref, encb_ref, a_ref):
    x0 = x_ref[...] - bpre_ref[...][None, :]
    acc = jax.lax.dot_general(
        x0, encW_ref[...], (((1,), (1,)), ((), ())),
        preferred_element_type=jnp.float32,
    )
    a_ref[...] = acc + encb_ref[...][None, :]


def _encode(x, b_pre, enc_W, enc_b):
    BB, LB = 1024, 2048
    grid = (D_LATENT // LB, BC // BB)
    return pl.pallas_call(
        _enc_body,
        grid=grid,
        in_specs=[
            pl.BlockSpec((BB, D_IN), lambda j, i: (i, 0)),
            pl.BlockSpec((D_IN,), lambda j, i: (0,)),
            pl.BlockSpec((LB, D_IN), lambda j, i: (j, 0)),
            pl.BlockSpec((LB,), lambda j, i: (j,)),
        ],
        out_specs=pl.BlockSpec((BB, LB), lambda j, i: (i, j)),
        out_shape=jax.ShapeDtypeStruct((BC, D_LATENT), jnp.float32),
    )(x, b_pre, enc_W, enc_b)


def _sc_body(a_hbm, thr_hbm, rowbuf, hist, cand, thrbuf, offbuf, sem0, sem1):
    wid = lax.axis_index("s") * NC + lax.axis_index("c")
    base = wid * RPW
    lane = lax.iota(jnp.int32, 16)
    ones16 = jnp.ones((16,), jnp.int32)
    zeros16 = jnp.zeros((16,), jnp.int32)
    mask_all = lane < 16
    lane_off = lane * 129

    @plsc.parallel_loop(0, 130, unroll=4)
    def _(k):
        hist[pl.ds(k * 16, 16)] = zeros16

    def process_row(buf_slot, row_idx, guess):
        glo = jnp.maximum(guess << 25, 1)
        ghi = jnp.where(guess < (NBIN - 1), (guess + 1) << 25,
                        jnp.int32(0x7FFFFFFF))

        def passab(v, off16):
            x = rowbuf[buf_slot, pl.ds(v * 16, 16)]
            bits = lax.bitcast_convert_type(x, jnp.int32)
            idx = lax.shift_right_logical(bits, 25) + lane_off
            plsc.addupdate_scatter(hist, [idx], ones16, mask=mask_all)
            m = jnp.logical_and(bits >= glo, bits < ghi)
            plsc.store_scatter(cand, [off16 + lane], bits, mask=m)
            return off16 + jnp.where(m, 16, 0)

        off16 = plsc.parallel_loop(0, NV, unroll=16, carry=zeros16)(passab)

        run = jnp.int32(0)
        bstar = jnp.int32(0)
        cnt_above = jnp.int32(0)
        found = jnp.bool_(False)
        for c in range(NBIN // 16 - 1, -1, -1):
            acc = zeros16
            for l in range(16):
                acc = acc + hist[pl.ds(l * 129 + c * 16, 16)]
                hist[pl.ds(l * 129 + c * 16, 16)] = zeros16
            s = jnp.sum(acc)
            rc = plsc.cumsum(lax.rev(acc, (0,)))
            tot = rc + run
            below = tot < K
            p_cnt = jnp.sum(below.astype(jnp.int32))
            hit = jnp.logical_and(jnp.logical_not(found),
                                  jnp.logical_and(run + s >= K, p_cnt < 16))
            d_here = c * 16 + 15 - p_cnt
            ca_here = jnp.maximum(run, jnp.max(jnp.where(below, tot, 0)))
            bstar = jnp.where(hit, d_here, bstar)
            cnt_above = jnp.where(hit, ca_here, cnt_above)
            found = jnp.logical_or(found, hit)
            run = run + s

        lo = jnp.maximum(bstar << 25, 1)
        hi = jnp.where(bstar < (NBIN - 1), (bstar + 1) << 25,
                       jnp.int32(0x7FFFFFFF))
        cnt_above = jnp.where(found, cnt_above, run)
        k_rem = K - cnt_above

        miss = bstar != guess

        @pl.when(miss)
        def _():
            def compact(v, off16b):
                x = rowbuf[buf_slot, pl.ds(v * 16, 16)]
                bits = lax.bitcast_convert_type(x, jnp.int32)
                m = jnp.logical_and(bits >= lo, bits < hi)
                plsc.store_scatter(cand, [off16b + lane], bits, mask=m)
                return off16b + jnp.where(m, 16, 0)

            off16b = plsc.parallel_loop(0, NV, unroll=16,
                                        carry=zeros16)(compact)
            offbuf[pl.ds(0, 16)] = off16b

        off16f = jnp.where(miss, offbuf[pl.ds(0, 16)], off16)
        valid_n = lax.shift_right_logical(off16f, 4)
        nmax = jnp.max(valid_n)
        k_rem_s = zeros16 + k_rem

        t = zeros16 + (bstar << 25)
        for b in range(24, -1, -1):
            candt = t | (1 << b)

            def count_vreg(i, acc):
                cvals = cand[pl.ds(i * 16, 16)]
                ok = jnp.logical_and(cvals >= candt, valid_n > i)
                return acc + plsc.all_reduce_population_count(ok)

            cnt = plsc.parallel_loop(0, nmax, unroll=2, carry=zeros16)(count_vreg)
            t = jnp.where(cnt >= k_rem_s, candt, t)

        tval = lax.bitcast_convert_type(jnp.maximum(t, 1), jnp.float32)
        plsc.store_scatter(thrbuf, [zeros16 + row_idx], tval, mask=lane == 0)
        return bstar

    pltpu.async_copy(a_hbm.at[base], rowbuf.at[0], sem0)
    pltpu.async_copy(a_hbm.at[base + 1], rowbuf.at[1], sem1)

    @pl.loop(0, RPW // 2, init_carry=jnp.int32(32))
    def _(g, guess):
        r0 = base + 2 * g
        pltpu.make_async_copy(a_hbm.at[r0], rowbuf.at[0], sem0).wait()
        b0 = process_row(0, 2 * g, guess)

        @pl.when(g < RPW // 2 - 1)
        def _():
            pltpu.async_copy(a_hbm.at[r0 + 2], rowbuf.at[0], sem0)

        pltpu.make_async_copy(a_hbm.at[r0 + 1], rowbuf.at[1], sem1).wait()
        b1 = process_row(1, 2 * g + 1, b0)

        @pl.when(g < RPW // 2 - 1)
        def _():
            pltpu.async_copy(a_hbm.at[r0 + 3], rowbuf.at[1], sem1)

        return b1

    pltpu.sync_copy(thrbuf, thr_hbm.at[pl.ds(base, RPW)])


def _sc_select(a):
    mesh = plsc.VectorSubcoreMesh(
        core_axis_name="c", subcore_axis_name="s",
        num_cores=NC, num_subcores=NW // NC,
    )
    kern = pl.kernel(
        _sc_body,
        out_type=jax.ShapeDtypeStruct((BC,), jnp.float32),
        mesh=mesh,
        compiler_params=pltpu.CompilerParams(needs_layout_passes=False),
        scratch_types=[
            pltpu.VMEM((2, D_LATENT), jnp.float32),
            pltpu.VMEM((2080,), jnp.int32),
            pltpu.VMEM((D_LATENT + 16,), jnp.int32),
            pltpu.VMEM((RPW,), jnp.float32),
            pltpu.VMEM((16,), jnp.int32),
            pltpu.SemaphoreType.DMA,
            pltpu.SemaphoreType.DMA,
        ],
    )
    return kern(a)


def _dec_body(a_ref, thr_ref, decW_ref, decb_ref, o_ref, z_ref):
    j = pl.program_id(1)
    z = jnp.where(a_ref[...] >= thr_ref[...], a_ref[...], 0.0)
    z_ref[...] = z
    acc = jax.lax.dot_general(
        z, decW_ref[...], (((1,), (1,)), ((), ())),
        preferred_element_type=jnp.float32,
    )

    @pl.when(j == 0)
    def _():
        o_ref[...] = acc + decb_ref[...][None, :]

    @pl.when(j > 0)
    def _():
        o_ref[...] = o_ref[...] + acc


def _decode(a, thr, dec_W, dec_b):
    BB, LB = 1024, 1024
    grid = (BC // BB, D_LATENT // LB)
    return pl.pallas_call(
        _dec_body,
        grid=grid,
        in_specs=[
            pl.BlockSpec((BB, LB), lambda i, j: (i, j)),
            pl.BlockSpec((BB, 1), lambda i, j: (i, 0)),
            pl.BlockSpec((D_IN, LB), lambda i, j: (0, j)),
            pl.BlockSpec((D_IN,), lambda i, j: (0,)),
        ],
        out_specs=[
            pl.BlockSpec((BB, D_IN), lambda i, j: (i, 0)),
            pl.BlockSpec((BB, LB), lambda i, j: (i, j)),
        ],
        out_shape=[
            jax.ShapeDtypeStruct((BC, D_IN), jnp.float32),
            jax.ShapeDtypeStruct((BC, D_LATENT), jnp.float32),
        ],
    )(a, thr, dec_W, dec_b)


def kernel(x, b_pre, enc_W, enc_b, dec_W, dec_b):
    a_c = []
    thr_c = []
    for i in range(NCHUNK):
        a_c.append(_encode(x[i * BC:(i + 1) * BC], b_pre, enc_W, enc_b))
        thr_c.append(_sc_select(a_c[i]))
    outs = [_decode(a_c[i], thr_c[i].reshape(BC, 1), dec_W, dec_b)
            for i in range(NCHUNK)]
    x_hat = jnp.concatenate([o[0] for o in outs], axis=0)
    z = jnp.concatenate([o[1] for o in outs], axis=0)
    a = jnp.concatenate(a_c, axis=0)
    return (x_hat, z, a)
